# baseline (speedup 1.0000x reference)
import functools

import jax
import jax.numpy as jnp
from jax import lax
from jax.experimental import pallas as pl
from jax.experimental.pallas import tpu as pltpu
from jax.experimental.pallas import tpu_sc as plsc

DIM = 16
LANES = 16


@functools.lru_cache(maxsize=None)
def _make_kernel(total_rows):
    info = plsc.get_sparse_core_info()
    nw = info.num_cores * info.num_subcores
    n = total_rows // nw
    g = 128
    kg = 8
    chunk = g * kg
    nchunk = n // chunk
    assert n % chunk == 0 and total_rows % nw == 0

    mesh = plsc.VectorSubcoreMesh(core_axis_name="c", subcore_axis_name="s")

    @functools.partial(
        pl.kernel,
        mesh=mesh,
        compiler_params=pltpu.CompilerParams(
            needs_layout_passes=False, use_tc_tiling_on_sc=False),
        out_type=jax.ShapeDtypeStruct((total_rows, DIM), jnp.float32),
        scratch_types=[
            pltpu.VMEM((n,), jnp.int32),
            pltpu.VMEM((chunk, DIM), jnp.float32),
            pltpu.SemaphoreType.DMA,
        ],
    )
    def k(idx_hbm, table_hbm, out_hbm, idx_v, rows_v, sem):
        wid = lax.axis_index("s") * info.num_cores + lax.axis_index("c")
        base = wid * n
        pltpu.sync_copy(idx_hbm.at[pl.ds(base, n)], idx_v)

        def chunk_body(c, carry):
            cbase = c * chunk
            copies = []
            for b in range(kg):
                copies.append(pltpu.async_copy(
                    table_hbm.at[idx_v.at[pl.ds(cbase + b * g, g)]],
                    rows_v.at[pl.ds(b * g, g), :],
                    sem))
            for cp in copies:
                cp.wait()

            iota = lax.iota(jnp.int32, LANES)
            zeros = jnp.zeros((LANES,), jnp.float32)

            def group_body(j, inner):
                keys = idx_v[pl.ds(cbase + j * LANES, LANES)]
                m = keys == 0
                rvec = j * LANES + iota
                for col in range(DIM):
                    plsc.store_scatter(
                        rows_v,
                        [rvec, jnp.full((LANES,), col, jnp.int32)],
                        zeros, mask=m)
                return inner

            lax.fori_loop(0, chunk // LANES, group_body, 0)
            pltpu.sync_copy(rows_v, out_hbm.at[pl.ds(base + cbase, chunk), :])
            return carry

        lax.fori_loop(0, nchunk, chunk_body, 0)

    return k


def kernel(input, table):
    b, w = input.shape
    idx = input.reshape(-1).astype(jnp.int32)
    out = _make_kernel(b * w)(idx, table)
    return out.reshape(b, w, DIM)
